# Initial kernel scaffold; baseline (speedup 1.0000x reference)
import jax
import jax.numpy as jnp
from jax.experimental import pallas as pl


def kernel(edge_index, x_s, x_t, W):
    raise NotImplementedError("write your pallas kernel here")


# speedup vs baseline: 21.0949x; 21.0949x over previous
import functools

import jax
import jax.numpy as jnp
from jax import lax
from jax.experimental import pallas as pl
from jax.experimental.pallas import tpu as pltpu
from jax.experimental.pallas import tpu_sc as plsc

N_SRC = 5000
N_TGT = 5000
N_EDGE = 320000
D = 128
TW = 144
ONES_COL = 128

NC = 2
NS = 16
NW = NC * NS
CHUNK = 128
NCHUNK = N_EDGE // CHUNK
CPT = -(-NCHUNK // NW)

MM_BLK = 1000
CB_BLK = 1000


def _mm_body(x_ref, w_ref, o_ref):
    acc = jnp.dot(x_ref[...], w_ref[...], preferred_element_type=jnp.float32,
                  precision=lax.Precision.HIGHEST)
    col = lax.broadcasted_iota(jnp.int32, acc.shape, 1)
    o_ref[...] = acc + (col == ONES_COL).astype(jnp.float32)


_matmul = pl.pallas_call(
    _mm_body,
    grid=((N_SRC + N_TGT) // MM_BLK,),
    in_specs=[
        pl.BlockSpec((MM_BLK, D), lambda i: (i, 0)),
        pl.BlockSpec((D, TW), lambda i: (0, 0)),
    ],
    out_specs=pl.BlockSpec((MM_BLK, TW), lambda i: (i, 0)),
    out_shape=jax.ShapeDtypeStruct((N_SRC + N_TGT, TW), jnp.float32),
)


@functools.cache
def _make_edge_aggregate():
    mesh = plsc.VectorSubcoreMesh(
        core_axis_name="c", subcore_axis_name="s",
        num_cores=NC, num_subcores=NS)
    return pl.kernel(
        _edge_aggregate_body,
        out_type=jax.ShapeDtypeStruct((NC, N_TGT, TW), jnp.float32),
        mesh=mesh,
        scratch_types=[
            pltpu.VMEM((CHUNK,), jnp.int32),
            pltpu.VMEM((CHUNK,), jnp.int32),
            pltpu.VMEM((CHUNK, TW), jnp.float32),
            pltpu.VMEM_SHARED((N_TGT, TW), jnp.float32),
            pltpu.SemaphoreType.DMA,
        ],
        compiler_params=pltpu.CompilerParams(use_tc_tiling_on_sc=False),
    )


def _edge_aggregate_body(table, src, dst, zeros, out, src_v, dst_v, rows_v,
                         acc_sh, sem):
    c = lax.axis_index("c")
    s = lax.axis_index("s")
    wid = s * NC + c

    @pl.when(s == 0)
    def _():
        pltpu.sync_copy(zeros, acc_sh)

    plsc.subcore_barrier()

    def body(k, carry):
        cid = wid + k * NW

        @pl.when(cid < NCHUNK)
        def _():
            off = cid * CHUNK
            pltpu.sync_copy(src.at[pl.ds(off, CHUNK)], src_v)
            pltpu.sync_copy(dst.at[pl.ds(off, CHUNK)], dst_v)
            pltpu.async_copy(table.at[src_v], rows_v, sem).wait()
            pltpu.sync_copy(rows_v, acc_sh.at[dst_v], add=True)

        return carry

    lax.fori_loop(0, CPT, body, 0)

    plsc.subcore_barrier()

    @pl.when(s == 0)
    def _():
        pltpu.sync_copy(acc_sh, out.at[c])


def _combine_body(agg_ref, xs_ref, xt_ref, os_ref, ot_ref):
    a = agg_ref[0] + agg_ref[1]
    feat = a[:, :D]
    deg = a[:, ONES_COL] + 1.0
    dis = 1.0 / jnp.sqrt(deg)
    ot = dis[:, None] * feat + (dis * dis)[:, None] * xt_ref[:, :D]
    ot_ref[...] = jnp.maximum(ot, 0.0)
    os_ref[...] = jnp.maximum(xs_ref[:, :D], 0.0)


_combine = pl.pallas_call(
    _combine_body,
    grid=(N_TGT // CB_BLK,),
    in_specs=[
        pl.BlockSpec((NC, CB_BLK, TW), lambda i: (0, i, 0)),
        pl.BlockSpec((CB_BLK, TW), lambda i: (i, 0)),
        pl.BlockSpec((CB_BLK, TW), lambda i: (i + N_SRC // CB_BLK, 0)),
    ],
    out_specs=[
        pl.BlockSpec((CB_BLK, D), lambda i: (i, 0)),
        pl.BlockSpec((CB_BLK, D), lambda i: (i, 0)),
    ],
    out_shape=[
        jax.ShapeDtypeStruct((N_SRC, D), jnp.float32),
        jax.ShapeDtypeStruct((N_TGT, D), jnp.float32),
    ],
)


def kernel(edge_index, x_s, x_t, W):
    x = jnp.concatenate([x_s, x_t], axis=0)
    w_ext = jnp.pad(W, ((0, 0), (0, TW - D)))
    xw = _matmul(x, w_ext)
    src = edge_index[0]
    dst = edge_index[1]
    zeros = jnp.zeros((N_TGT, TW), jnp.float32)
    agg = _make_edge_aggregate()(xw, src, dst, zeros)
    out_s, out_t = _combine(agg, xw, xw)
    return out_s, out_t
